# baseline (speedup 1.0000x reference)
import functools

import jax
import jax.numpy as jnp
from jax import lax
from jax.experimental import pallas as pl
from jax.experimental.pallas import tpu as pltpu
from jax.experimental.pallas import tpu_sc as plsc

B, S, D = 1024, 200, 128
N = B * S
NC, NS = 2, 16
NW = NC * NS
PER_W = N // NW
CHUNK = 128
STEPS = PER_W // CHUNK
NBUF = 5
OUTER = STEPS // NBUF
SKEW = 1

_mesh = plsc.VectorSubcoreMesh(core_axis_name="c", subcore_axis_name="s")


@functools.partial(
    pl.kernel,
    mesh=_mesh,
    out_type=jax.ShapeDtypeStruct((N, D), jnp.float32),
    scratch_types=[
        pltpu.VMEM((STEPS, CHUNK), jnp.int32),
        pltpu.VMEM((NBUF, CHUNK, D), jnp.float32),
    ]
    + [pltpu.SemaphoreType.DMA] * (2 * NBUF),
)
def _gather(idx_hbm, table_hbm, out_hbm, idx_v, bufs, *sems):
    gsem = sems[:NBUF]
    ssem = sems[NBUF:]
    wid = lax.axis_index("s") * NC + lax.axis_index("c")
    pltpu.sync_copy(idx_hbm.at[wid], idx_v)

    out_base = wid * PER_W

    for b in range(SKEW):
        pltpu.async_copy(table_hbm.at[idx_v.at[b]], bufs.at[b], gsem[b])

    def outer(t, _):
        for b in range(NBUF):
            i = t * NBUF + b
            pltpu.make_async_copy(
                table_hbm.at[idx_v.at[i]], bufs.at[b], gsem[b]
            ).wait()
            pltpu.async_copy(
                bufs.at[b], out_hbm.at[pl.ds(out_base + i * CHUNK, CHUNK)], ssem[b]
            )
            nb = (b + SKEW) % NBUF
            n = i + SKEW

            @pl.when(n < STEPS)
            def _():
                @pl.when(n >= NBUF)
                def _():
                    pltpu.make_async_copy(
                        bufs.at[nb],
                        out_hbm.at[pl.ds(out_base + (n - NBUF) * CHUNK, CHUNK)],
                        ssem[nb],
                    ).wait()

                pltpu.async_copy(table_hbm.at[idx_v.at[n]], bufs.at[nb], gsem[nb])

        return 0

    lax.fori_loop(0, OUTER, outer, 0)

    for j in range(STEPS - NBUF + SKEW, STEPS):
        b = j % NBUF
        pltpu.make_async_copy(
            bufs.at[b], out_hbm.at[pl.ds(out_base + j * CHUNK, CHUNK)], ssem[b]
        ).wait()


def kernel(X_word, X_char, word_embed_W):
    del X_char
    idx2d = X_word.reshape(NW, STEPS, CHUNK)
    out = _gather(idx2d, word_embed_W)
    return out.reshape(B, S, D)


# speedup vs baseline: 1.2209x; 1.2209x over previous
import functools

import jax
import jax.numpy as jnp
from jax import lax
from jax.experimental import pallas as pl
from jax.experimental.pallas import tpu as pltpu
from jax.experimental.pallas import tpu_sc as plsc

B, S, D = 1024, 200, 128
N = B * S
NC, NS = 2, 16
NW = NC * NS
PER_W = N // NW
CHUNK = 128
STEPS = PER_W // CHUNK
NBUF = 5
OUTER = STEPS // NBUF
SKEW = 2

_mesh = plsc.VectorSubcoreMesh(core_axis_name="c", subcore_axis_name="s")


@functools.partial(
    pl.kernel,
    mesh=_mesh,
    out_type=jax.ShapeDtypeStruct((N, D), jnp.float32),
    scratch_types=[
        pltpu.VMEM((STEPS, CHUNK), jnp.int32),
        pltpu.VMEM((NBUF, CHUNK, D), jnp.float32),
    ]
    + [pltpu.SemaphoreType.DMA] * (2 * NBUF),
)
def _gather(idx_hbm, table_hbm, out_hbm, idx_v, bufs, *sems):
    gsem = sems[:NBUF]
    ssem = sems[NBUF:]
    wid = lax.axis_index("s") * NC + lax.axis_index("c")
    pltpu.sync_copy(idx_hbm.at[wid, pl.ds(0, 8)], idx_v.at[pl.ds(0, 8)])

    out_base = wid * PER_W

    for b in range(SKEW):
        pltpu.async_copy(table_hbm.at[idx_v.at[b]], bufs.at[b], gsem[b])

    pltpu.sync_copy(
        idx_hbm.at[wid, pl.ds(8, STEPS - 8)], idx_v.at[pl.ds(8, STEPS - 8)]
    )

    def outer(t, _):
        for b in range(NBUF):
            i = t * NBUF + b
            pltpu.make_async_copy(
                table_hbm.at[idx_v.at[i]], bufs.at[b], gsem[b]
            ).wait()
            pltpu.async_copy(
                bufs.at[b], out_hbm.at[pl.ds(out_base + i * CHUNK, CHUNK)], ssem[b]
            )
            nb = (b + SKEW) % NBUF
            n = i + SKEW

            @pl.when(n < STEPS)
            def _():
                @pl.when(n >= NBUF)
                def _():
                    pltpu.make_async_copy(
                        bufs.at[nb],
                        out_hbm.at[pl.ds(out_base + (n - NBUF) * CHUNK, CHUNK)],
                        ssem[nb],
                    ).wait()

                pltpu.async_copy(table_hbm.at[idx_v.at[n]], bufs.at[nb], gsem[nb])

        return 0

    lax.fori_loop(0, OUTER, outer, 0)

    for j in range(STEPS - NBUF + SKEW, STEPS):
        b = j % NBUF
        pltpu.make_async_copy(
            bufs.at[b], out_hbm.at[pl.ds(out_base + j * CHUNK, CHUNK)], ssem[b]
        ).wait()


def kernel(X_word, X_char, word_embed_W):
    del X_char
    idx2d = X_word.reshape(NW, STEPS, CHUNK)
    out = _gather(idx2d, word_embed_W)
    return out.reshape(B, S, D)
